# baseline (speedup 1.0000x reference)
import functools

import jax
import jax.numpy as jnp
from jax.experimental import pallas as pl
from jax.experimental.pallas import tpu as pltpu

_BATCH, _ROW, _COL = 16, 4096, 256
_NS = 64


_KCH = 4
_CHROW = _ROW // _KCH


def _matmul_body(param_ref, x_ref, out_ref, m_ref, acc_ref):
    b = pl.program_id(0)
    k = pl.program_id(1)

    @pl.when((b == 0) & (k == 0))
    def _build_mask():
        idx = jax.lax.broadcasted_iota(jnp.int32, (_NS, _ROW), 1)
        s0 = param_ref[:, 0:1]
        s1 = param_ref[:, 1:2]
        mask = (idx >= s0) & (idx < s1)
        m_ref[...] = mask.astype(jnp.bfloat16)

    x = x_ref[0].astype(jnp.bfloat16)
    m = m_ref[:, pl.ds(k * _CHROW, _CHROW)]
    part = jax.lax.dot(m, x, preferred_element_type=jnp.float32)

    @pl.when(k == 0)
    def _init():
        acc_ref[...] = part

    @pl.when(k > 0)
    def _acc():
        acc_ref[...] += part

    @pl.when(k == _KCH - 1)
    def _emit():
        out_ref[0] = acc_ref[...]


def kernel(input, slice_param):
    out = pl.pallas_call(
        _matmul_body,
        grid=(_BATCH, _KCH),
        in_specs=[
            pl.BlockSpec((_NS, 2), lambda b, k: (0, 0)),
            pl.BlockSpec((1, _CHROW, _COL), lambda b, k: (b, k, 0)),
        ],
        out_specs=pl.BlockSpec((1, _NS, _COL), lambda b, k: (b, 0, 0)),
        out_shape=jax.ShapeDtypeStruct((_BATCH, _NS, _COL), jnp.float32),
        scratch_shapes=[
            pltpu.VMEM((_NS, _ROW), jnp.bfloat16),
            pltpu.VMEM((_NS, _COL), jnp.float32),
        ],
    )(slice_param, input)
    return out.reshape(_BATCH, _NS * _COL)


# speedup vs baseline: 2.1257x; 2.1257x over previous
import functools

import jax
import jax.numpy as jnp
from jax.experimental import pallas as pl
from jax.experimental.pallas import tpu as pltpu

_BATCH, _ROW, _COL = 16, 4096, 256
_NS = 64


def _matmul_body(param_ref, x_ref, out_ref, m_ref):
    b = pl.program_id(0)

    @pl.when(b == 0)
    def _build_mask():
        idx = jax.lax.broadcasted_iota(jnp.int32, (_NS, _ROW), 1)
        s0 = param_ref[:, 0:1]
        s1 = param_ref[:, 1:2]
        mask = (idx >= s0) & (idx < s1)
        m_ref[...] = mask.astype(jnp.bfloat16)

    for i in range(_BPB):
        x = x_ref[i].astype(jnp.bfloat16)
        out_ref[i] = jax.lax.dot(
            m_ref[...], x, preferred_element_type=jnp.float32
        )


_BPB = 2


def kernel(input, slice_param):
    out = pl.pallas_call(
        _matmul_body,
        grid=(_BATCH // _BPB,),
        in_specs=[
            pl.BlockSpec((_NS, 2), lambda b: (0, 0)),
            pl.BlockSpec((_BPB, _ROW, _COL), lambda b: (b, 0, 0)),
        ],
        out_specs=pl.BlockSpec((_BPB, _NS, _COL), lambda b: (b, 0, 0)),
        out_shape=jax.ShapeDtypeStruct((_BATCH, _NS, _COL), jnp.float32),
        scratch_shapes=[pltpu.VMEM((_NS, _ROW), jnp.bfloat16)],
    )(slice_param, input)
    return out.reshape(_BATCH, _NS * _COL)
